# baseline (speedup 1.0000x reference)
import functools

import jax
import jax.numpy as jnp
from jax import lax
from jax.experimental import pallas as pl
from jax.experimental.pallas import tpu as pltpu
from jax.experimental.pallas import tpu_sc as plsc

N_NODES = 10000
NP = 10240
D = 128
NE = 320000
NC, NS = 2, 16
NW = NC * NS
EW = 10240
KB = EW // 128
BS = 64
KBS = EW // BS
NE_PAD = EW * NW
ROWS_PER_TILE = NP // NS
BM = 1024


def _mm2_body(x_ref, wl_ref, wr_ref, b_ref, y_ref, z_ref):
    x = x_ref[...]
    y_ref[...] = jnp.dot(x, wl_ref[...], preferred_element_type=jnp.float32)
    z_ref[...] = (jnp.dot(x, wr_ref[...], preferred_element_type=jnp.float32)
                  + b_ref[...])


def _dual_matmul(x, wl, wr, b):
    bm = N_NODES // 10
    return pl.pallas_call(
        _mm2_body,
        grid=(10,),
        in_specs=[
            pl.BlockSpec((bm, D), lambda i: (i, 0)),
            pl.BlockSpec((D, D), lambda i: (0, 0)),
            pl.BlockSpec((D, D), lambda i: (0, 0)),
            pl.BlockSpec((1, D), lambda i: (0, 0)),
        ],
        out_specs=[pl.BlockSpec((bm, D), lambda i: (i, 0)),
                   pl.BlockSpec((bm, D), lambda i: (i, 0))],
        out_shape=[jax.ShapeDtypeStruct((NP, D), jnp.float32),
                   jax.ShapeDtypeStruct((NP, D), jnp.float32)],
    )(x, wl, wr, b.reshape(1, D))


def _fused_body(p_ref, z_ref, wl_ref, wr_ref, b_ref, y_ref, z2_ref):
    h = jnp.maximum(p_ref[0] + p_ref[1] + z_ref[...], 0.0)
    y_ref[...] = jnp.dot(h, wl_ref[...], preferred_element_type=jnp.float32)
    z2_ref[...] = (jnp.dot(h, wr_ref[...], preferred_element_type=jnp.float32)
                   + b_ref[...])


def _relu_sum_matmul(p, z1, wl, wr, b):
    return pl.pallas_call(
        _fused_body,
        grid=(NP // BM,),
        in_specs=[
            pl.BlockSpec((2, BM, D), lambda i: (0, i, 0)),
            pl.BlockSpec((BM, D), lambda i: (i, 0)),
            pl.BlockSpec((D, D), lambda i: (0, 0)),
            pl.BlockSpec((D, D), lambda i: (0, 0)),
            pl.BlockSpec((1, D), lambda i: (0, 0)),
        ],
        out_specs=[pl.BlockSpec((BM, D), lambda i: (i, 0)),
                   pl.BlockSpec((BM, D), lambda i: (i, 0))],
        out_shape=[jax.ShapeDtypeStruct((NP, D), jnp.float32),
                   jax.ShapeDtypeStruct((NP, D), jnp.float32)],
    )(p, z1, wl, wr, b.reshape(1, D))


def _sum_body(q_ref, z_ref, o_ref):
    o_ref[...] = q_ref[0] + q_ref[1] + z_ref[...]


def _partial_sum(q, z2):
    return pl.pallas_call(
        _sum_body,
        grid=(NP // BM,),
        in_specs=[
            pl.BlockSpec((2, BM, D), lambda i: (0, i, 0)),
            pl.BlockSpec((BM, D), lambda i: (i, 0)),
        ],
        out_specs=pl.BlockSpec((BM, D), lambda i: (i, 0)),
        out_shape=jax.ShapeDtypeStruct((NP, D), jnp.float32),
    )(q, z2)


_MESH = plsc.VectorSubcoreMesh(core_axis_name="c", subcore_axis_name="s")


@functools.partial(
    pl.kernel,
    out_type=jax.ShapeDtypeStruct((NC, NP, D), jnp.float32),
    mesh=_MESH,
    compiler_params=pltpu.CompilerParams(needs_layout_passes=False),
    scratch_types=[
        pltpu.VMEM_SHARED((NP, D), jnp.float32),
        pltpu.VMEM((KBS // 2, BS), jnp.int32),
        pltpu.VMEM((KBS // 2, BS), jnp.int32),
        pltpu.VMEM((BS, D), jnp.float32),
        pltpu.VMEM((BS, D), jnp.float32),
        pltpu.SemaphoreType.DMA,
        pltpu.SemaphoreType.DMA,
    ],
)
def _scatter_kernel(y_hbm, src_hbm, dst_hbm, out_hbm, acc, sidx, didx,
                    rows_a, rows_b, sem_a, sem_b):
    cid = lax.axis_index("c")
    sid = lax.axis_index("s")
    wid = cid * NS + sid

    def _zero(k, _):
        rows_a[k // 8, pl.ds((k % 8) * 16, 16)] = jnp.zeros((16,), jnp.float32)
        return 0
    lax.fori_loop(0, BS * 8, _zero, 0)
    for i in range(ROWS_PER_TILE // BS):
        pltpu.sync_copy(rows_a,
                        acc.at[pl.ds(sid * ROWS_PER_TILE + i * BS, BS)])

    plsc.subcore_barrier()

    def _start(j, buf, sem):
        pltpu.async_copy(y_hbm.at[sidx.at[j]], buf, sem)

    def _wait(j, buf, sem):
        pltpu.make_async_copy(y_hbm.at[sidx.at[j]], buf, sem).wait()

    hkb = KBS // 2
    for ph in range(2):
        pltpu.sync_copy(src_hbm.at[wid, pl.ds(ph * hkb, hkb)], sidx)
        pltpu.sync_copy(dst_hbm.at[wid, pl.ds(ph * hkb, hkb)], didx)
        _start(0, rows_a, sem_a)

        def _body(k, _):
            j0 = 2 * k
            j1 = j0 + 1
            _start(j1, rows_b, sem_b)
            _wait(j0, rows_a, sem_a)
            pltpu.sync_copy(rows_a, acc.at[didx.at[j0]], add=True)

            @pl.when(k < hkb // 2 - 1)
            def _():
                _start(j1 + 1, rows_a, sem_a)
            _wait(j1, rows_b, sem_b)
            pltpu.sync_copy(rows_b, acc.at[didx.at[j1]], add=True)
            return 0
        lax.fori_loop(0, hkb // 2, _body, 0)

    plsc.subcore_barrier()
    pltpu.sync_copy(acc.at[pl.ds(sid * ROWS_PER_TILE, ROWS_PER_TILE)],
                    out_hbm.at[cid, pl.ds(sid * ROWS_PER_TILE, ROWS_PER_TILE)])


@functools.partial(
    pl.kernel,
    out_type=jax.ShapeDtypeStruct((NE_PAD,), jnp.float32),
    mesh=_MESH,
    compiler_params=pltpu.CompilerParams(needs_layout_passes=False),
    scratch_types=[
        pltpu.VMEM((KB, 128), jnp.int32),
        pltpu.VMEM((KB, 128), jnp.int32),
        pltpu.VMEM((128, D), jnp.float32),
        pltpu.VMEM((128, D), jnp.float32),
        pltpu.VMEM((128, D), jnp.float32),
        pltpu.VMEM((128, D), jnp.float32),
        pltpu.VMEM((256,), jnp.float32),
        pltpu.VMEM((256,), jnp.float32),
        pltpu.VMEM((128,), jnp.float32),
        pltpu.VMEM((128,), jnp.float32),
        pltpu.SemaphoreType.DMA,
        pltpu.SemaphoreType.DMA,
        pltpu.SemaphoreType.DMA,
        pltpu.SemaphoreType.DMA,
    ],
)
def _edgedot_kernel(h_hbm, src_hbm, dst_hbm, out_hbm, sidx, didx, srows_a,
                    drows_a, srows_b, drows_b, tbuf_a, tbuf_b, obuf_a, obuf_b,
                    sem_a, sem_b, sem_oa, sem_ob):
    cid = lax.axis_index("c")
    sid = lax.axis_index("s")
    wid = cid * NS + sid
    pltpu.sync_copy(src_hbm.at[wid], sidx)
    pltpu.sync_copy(dst_hbm.at[wid], didx)
    col0 = lax.iota(jnp.int32, 16) * 16

    def _start(j, sbuf, dbuf, sem):
        pltpu.async_copy(h_hbm.at[sidx.at[j]], sbuf, sem)
        pltpu.async_copy(h_hbm.at[didx.at[j]], dbuf, sem)

    def _wait(j, sbuf, dbuf, sem):
        pltpu.make_async_copy(h_hbm.at[sidx.at[j]], sbuf, sem).wait()
        pltpu.make_async_copy(h_hbm.at[didx.at[j]], dbuf, sem).wait()

    def _owait(j, ob, sem):
        pltpu.make_async_copy(
            ob, out_hbm.at[pl.ds(wid * EW + j * 128, 128)], sem).wait()

    def _products(e0, srows, drows, tref):
        for e16 in range(16):
            e = e0 + e16
            p = [srows[e, pl.ds(c * 16, 16)] * drows[e, pl.ds(c * 16, 16)]
                 for c in range(8)]
            q = [p[0] + p[1], p[2] + p[3], p[4] + p[5], p[6] + p[7]]
            tref[pl.ds(e16 * 16, 16)] = (q[0] + q[1]) + (q[2] + q[3])

    def _reduce(g, tref, ob):
        v = [plsc.load_gather(tref, [col0 + l]) for l in range(16)]
        for step in (8, 4, 2, 1):
            v = [v[i] + v[i + step] for i in range(step)]
        ob[pl.ds(g * 16, 16)] = 1.0 / (1.0 + jnp.exp(-v[0]))

    def _compute(j, srows, drows, ob, sem_o):
        for gp in range(4):
            g0 = 2 * gp
            _products(g0 * 16, srows, drows, tbuf_a)
            _products(g0 * 16 + 16, srows, drows, tbuf_b)
            _reduce(g0, tbuf_a, ob)
            _reduce(g0 + 1, tbuf_b, ob)
        pltpu.async_copy(ob, out_hbm.at[pl.ds(wid * EW + j * 128, 128)],
                         sem_o)

    _start(0, srows_a, drows_a, sem_a)

    def _body(k, _):
        j0 = 2 * k
        j1 = j0 + 1
        _start(j1, srows_b, drows_b, sem_b)
        _wait(j0, srows_a, drows_a, sem_a)

        @pl.when(k > 0)
        def _():
            _owait(j0 - 2, obuf_a, sem_oa)
        _compute(j0, srows_a, drows_a, obuf_a, sem_oa)

        @pl.when(k < KB // 2 - 1)
        def _():
            _start(j1 + 1, srows_a, drows_a, sem_a)
        _wait(j1, srows_b, drows_b, sem_b)

        @pl.when(k > 0)
        def _():
            _owait(j1 - 2, obuf_b, sem_ob)
        _compute(j1, srows_b, drows_b, obuf_b, sem_ob)
        return 0
    lax.fori_loop(0, KB // 2, _body, 0)
    _owait(KB - 2, obuf_a, sem_oa)
    _owait(KB - 1, obuf_b, sem_ob)


def _prep_edges(idx2, pad, nb):
    s = jnp.concatenate([idx2[0], pad]).reshape(NW, nb, EW // nb)
    d = jnp.concatenate([idx2[1], pad]).reshape(NW, nb, EW // nb)
    return s, d


def kernel(Features, A, E, Wl1, Wr1, b1, Wl2, Wr2, b2):
    pad = N_NODES + (jnp.arange(NE_PAD - NE, dtype=jnp.int32)
                     % (NP - N_NODES))
    a_src, a_dst = _prep_edges(A, pad, KBS)
    e_src, e_dst = _prep_edges(E, pad, KB)

    y1, z1 = _dual_matmul(Features, Wl1, Wr1, b1)
    p = _scatter_kernel(y1, a_src, a_dst)
    y2, z2 = _relu_sum_matmul(p, z1, Wl2, Wr2, b2)
    q = _scatter_kernel(y2, a_src, a_dst)
    h2 = _partial_sum(q, z2)
    out = _edgedot_kernel(h2, e_src, e_dst)
    return out[:NE]


# speedup vs baseline: 1.5897x; 1.5897x over previous
import functools

import jax
import jax.numpy as jnp
from jax import lax
from jax.experimental import pallas as pl
from jax.experimental.pallas import tpu as pltpu
from jax.experimental.pallas import tpu_sc as plsc

N_NODES = 10000
NP = 10240
D = 128
NE = 320000
NC, NS = 2, 16
NW = NC * NS
EW = 10240
KB = EW // 128
BS = 64
KBS = EW // BS
NE_PAD = EW * NW
ROWS_PER_TILE = NP // NS
BM = 1024


def _mm2_body(x_ref, wl_ref, wr_ref, b_ref, y_ref, z_ref):
    x = x_ref[...]
    y_ref[...] = jnp.dot(x, wl_ref[...], preferred_element_type=jnp.float32)
    z_ref[...] = (jnp.dot(x, wr_ref[...], preferred_element_type=jnp.float32)
                  + b_ref[...])


def _dual_matmul(x, wl, wr, b):
    bm = N_NODES // 10
    return pl.pallas_call(
        _mm2_body,
        grid=(10,),
        in_specs=[
            pl.BlockSpec((bm, D), lambda i: (i, 0)),
            pl.BlockSpec((D, D), lambda i: (0, 0)),
            pl.BlockSpec((D, D), lambda i: (0, 0)),
            pl.BlockSpec((1, D), lambda i: (0, 0)),
        ],
        out_specs=[pl.BlockSpec((bm, D), lambda i: (i, 0)),
                   pl.BlockSpec((bm, D), lambda i: (i, 0))],
        out_shape=[jax.ShapeDtypeStruct((NP, D), jnp.float32),
                   jax.ShapeDtypeStruct((NP, D), jnp.float32)],
    )(x, wl, wr, b.reshape(1, D))


def _fused_body(p_ref, z_ref, wl_ref, wr_ref, b_ref, y_ref, z2_ref):
    h = jnp.maximum(p_ref[0] + p_ref[1] + z_ref[...], 0.0)
    y_ref[...] = jnp.dot(h, wl_ref[...], preferred_element_type=jnp.float32)
    z2_ref[...] = (jnp.dot(h, wr_ref[...], preferred_element_type=jnp.float32)
                   + b_ref[...])


def _relu_sum_matmul(p, z1, wl, wr, b):
    return pl.pallas_call(
        _fused_body,
        grid=(NP // BM,),
        in_specs=[
            pl.BlockSpec((2, BM, D), lambda i: (0, i, 0)),
            pl.BlockSpec((BM, D), lambda i: (i, 0)),
            pl.BlockSpec((D, D), lambda i: (0, 0)),
            pl.BlockSpec((D, D), lambda i: (0, 0)),
            pl.BlockSpec((1, D), lambda i: (0, 0)),
        ],
        out_specs=[pl.BlockSpec((BM, D), lambda i: (i, 0)),
                   pl.BlockSpec((BM, D), lambda i: (i, 0))],
        out_shape=[jax.ShapeDtypeStruct((NP, D), jnp.float32),
                   jax.ShapeDtypeStruct((NP, D), jnp.float32)],
    )(p, z1, wl, wr, b.reshape(1, D))


def _sum_body(q_ref, z_ref, o_ref):
    o_ref[...] = q_ref[0] + q_ref[1] + z_ref[...]


def _partial_sum(q, z2):
    return pl.pallas_call(
        _sum_body,
        grid=(NP // BM,),
        in_specs=[
            pl.BlockSpec((2, BM, D), lambda i: (0, i, 0)),
            pl.BlockSpec((BM, D), lambda i: (i, 0)),
        ],
        out_specs=pl.BlockSpec((BM, D), lambda i: (i, 0)),
        out_shape=jax.ShapeDtypeStruct((NP, D), jnp.float32),
    )(q, z2)


_MESH = plsc.VectorSubcoreMesh(core_axis_name="c", subcore_axis_name="s")


@functools.partial(
    pl.kernel,
    out_type=jax.ShapeDtypeStruct((NC, NP, D), jnp.float32),
    mesh=_MESH,
    compiler_params=pltpu.CompilerParams(needs_layout_passes=False),
    scratch_types=[
        pltpu.VMEM_SHARED((NP, D), jnp.float32),
        pltpu.VMEM((KBS // 2, BS), jnp.int32),
        pltpu.VMEM((KBS // 2, BS), jnp.int32),
        pltpu.VMEM((BS, D), jnp.float32),
        pltpu.VMEM((BS, D), jnp.float32),
        pltpu.SemaphoreType.DMA,
        pltpu.SemaphoreType.DMA,
    ],
)
def _scatter_kernel(y_hbm, src_hbm, dst_hbm, out_hbm, acc, sidx, didx,
                    rows_a, rows_b, sem_a, sem_b):
    cid = lax.axis_index("c")
    sid = lax.axis_index("s")
    wid = cid * NS + sid

    def _zero(k, _):
        rows_a[k // 8, pl.ds((k % 8) * 16, 16)] = jnp.zeros((16,), jnp.float32)
        return 0
    lax.fori_loop(0, BS * 8, _zero, 0)
    for i in range(ROWS_PER_TILE // BS):
        pltpu.sync_copy(rows_a,
                        acc.at[pl.ds(sid * ROWS_PER_TILE + i * BS, BS)])

    plsc.subcore_barrier()

    def _start(j, buf, sem):
        pltpu.async_copy(y_hbm.at[sidx.at[j]], buf, sem)

    def _wait(j, buf, sem):
        pltpu.make_async_copy(y_hbm.at[sidx.at[j]], buf, sem).wait()

    hkb = KBS // 2
    for ph in range(2):
        pltpu.sync_copy(src_hbm.at[wid, pl.ds(ph * hkb, hkb)], sidx)
        pltpu.sync_copy(dst_hbm.at[wid, pl.ds(ph * hkb, hkb)], didx)
        _start(0, rows_a, sem_a)

        def _body(k, _):
            j0 = 2 * k
            j1 = j0 + 1
            _start(j1, rows_b, sem_b)
            _wait(j0, rows_a, sem_a)
            pltpu.sync_copy(rows_a, acc.at[didx.at[j0]], add=True)

            @pl.when(k < hkb // 2 - 1)
            def _():
                _start(j1 + 1, rows_a, sem_a)
            _wait(j1, rows_b, sem_b)
            pltpu.sync_copy(rows_b, acc.at[didx.at[j1]], add=True)
            return 0
        lax.fori_loop(0, hkb // 2, _body, 0)

    plsc.subcore_barrier()
    pltpu.sync_copy(acc.at[pl.ds(sid * ROWS_PER_TILE, ROWS_PER_TILE)],
                    out_hbm.at[cid, pl.ds(sid * ROWS_PER_TILE, ROWS_PER_TILE)])


@functools.partial(
    pl.kernel,
    out_type=jax.ShapeDtypeStruct((NE_PAD,), jnp.float32),
    mesh=_MESH,
    compiler_params=pltpu.CompilerParams(needs_layout_passes=False),
    scratch_types=[
        pltpu.VMEM((KB, 128), jnp.int32),
        pltpu.VMEM((KB, 128), jnp.int32),
        pltpu.VMEM((128, D), jnp.float32),
        pltpu.VMEM((128, D), jnp.float32),
        pltpu.VMEM((128, D), jnp.float32),
        pltpu.VMEM((128, D), jnp.float32),
        pltpu.VMEM((256,), jnp.float32),
        pltpu.VMEM((256,), jnp.float32),
        pltpu.VMEM((128,), jnp.float32),
        pltpu.VMEM((128,), jnp.float32),
        pltpu.SemaphoreType.DMA,
        pltpu.SemaphoreType.DMA,
        pltpu.SemaphoreType.DMA,
        pltpu.SemaphoreType.DMA,
    ],
)
def _edgedot_kernel(h_hbm, src_hbm, dst_hbm, out_hbm, sidx, didx, srows_a,
                    drows_a, srows_b, drows_b, tbuf_a, tbuf_b, obuf_a, obuf_b,
                    sem_a, sem_b, sem_oa, sem_ob):
    cid = lax.axis_index("c")
    sid = lax.axis_index("s")
    wid = cid * NS + sid
    pltpu.sync_copy(src_hbm.at[wid], sidx)
    pltpu.sync_copy(dst_hbm.at[wid], didx)
    col0 = lax.iota(jnp.int32, 16) * 16

    def _start(j, sbuf, dbuf, sem):
        pltpu.async_copy(h_hbm.at[sidx.at[j]], sbuf, sem)
        pltpu.async_copy(h_hbm.at[didx.at[j]], dbuf, sem)

    def _wait(j, sbuf, dbuf, sem):
        pltpu.make_async_copy(h_hbm.at[sidx.at[j]], sbuf, sem).wait()
        pltpu.make_async_copy(h_hbm.at[didx.at[j]], dbuf, sem).wait()

    def _owait(j, ob, sem):
        pltpu.make_async_copy(
            ob, out_hbm.at[pl.ds(wid * EW + j * 128, 128)], sem).wait()

    def _products(e0, srows, drows, tref):
        for e16 in range(16):
            e = e0 + e16
            p = [srows[e, pl.ds(c * 16, 16)] * drows[e, pl.ds(c * 16, 16)]
                 for c in range(8)]
            q = [p[0] + p[1], p[2] + p[3], p[4] + p[5], p[6] + p[7]]
            tref[pl.ds(e16 * 16, 16)] = (q[0] + q[1]) + (q[2] + q[3])

    def _reduce(g, tref, ob):
        v = [plsc.load_gather(tref, [col0 + l]) for l in range(16)]
        for step in (8, 4, 2, 1):
            v = [v[i] + v[i + step] for i in range(step)]
        ob[pl.ds(g * 16, 16)] = 1.0 / (1.0 + jnp.exp(-v[0]))

    def _compute(j, srows, drows, ob, sem_o):
        def _gpair(gp, _):
            g0 = 2 * gp
            _products(g0 * 16, srows, drows, tbuf_a)
            _products(g0 * 16 + 16, srows, drows, tbuf_b)
            _reduce(g0, tbuf_a, ob)
            _reduce(g0 + 1, tbuf_b, ob)
            return 0
        lax.fori_loop(0, 4, _gpair, 0)
        pltpu.async_copy(ob, out_hbm.at[pl.ds(wid * EW + j * 128, 128)],
                         sem_o)

    _start(0, srows_a, drows_a, sem_a)

    def _body(k, _):
        j0 = 2 * k
        j1 = j0 + 1
        _start(j1, srows_b, drows_b, sem_b)
        _wait(j0, srows_a, drows_a, sem_a)

        @pl.when(k > 0)
        def _():
            _owait(j0 - 2, obuf_a, sem_oa)
        _compute(j0, srows_a, drows_a, obuf_a, sem_oa)

        @pl.when(k < KB // 2 - 1)
        def _():
            _start(j1 + 1, srows_a, drows_a, sem_a)
        _wait(j1, srows_b, drows_b, sem_b)

        @pl.when(k > 0)
        def _():
            _owait(j1 - 2, obuf_b, sem_ob)
        _compute(j1, srows_b, drows_b, obuf_b, sem_ob)
        return 0
    lax.fori_loop(0, KB // 2, _body, 0)
    _owait(KB - 2, obuf_a, sem_oa)
    _owait(KB - 1, obuf_b, sem_ob)


def _prep_edges(idx2, pad, nb):
    s = jnp.concatenate([idx2[0], pad]).reshape(NW, nb, EW // nb)
    d = jnp.concatenate([idx2[1], pad]).reshape(NW, nb, EW // nb)
    return s, d


def kernel(Features, A, E, Wl1, Wr1, b1, Wl2, Wr2, b2):
    pad = N_NODES + (jnp.arange(NE_PAD - NE, dtype=jnp.int32)
                     % (NP - N_NODES))
    a_src, a_dst = _prep_edges(A, pad, KBS)
    e_src, e_dst = _prep_edges(E, pad, KB)

    y1, z1 = _dual_matmul(Features, Wl1, Wr1, b1)
    p = _scatter_kernel(y1, a_src, a_dst)
    y2, z2 = _relu_sum_matmul(p, z1, Wl2, Wr2, b2)
    q = _scatter_kernel(y2, a_src, a_dst)
    h2 = _partial_sum(q, z2)
    out = _edgedot_kernel(h2, e_src, e_dst)
    return out[:NE]


# speedup vs baseline: 1.6451x; 1.0348x over previous
import functools

import jax
import jax.numpy as jnp
from jax import lax
from jax.experimental import pallas as pl
from jax.experimental.pallas import tpu as pltpu
from jax.experimental.pallas import tpu_sc as plsc

N_NODES = 10000
NP = 10240
D = 128
NE = 320000
NC, NS = 2, 16
NW = NC * NS
EW = 10240
KB = EW // 128
BS = 80
KBS = EW // BS
NE_PAD = EW * NW
ROWS_PER_TILE = NP // NS
BM = 1024


def _mm2_body(x_ref, wl_ref, wr_ref, b_ref, y_ref, z_ref):
    x = x_ref[...]
    y_ref[...] = jnp.dot(x, wl_ref[...], preferred_element_type=jnp.float32)
    z_ref[...] = (jnp.dot(x, wr_ref[...], preferred_element_type=jnp.float32)
                  + b_ref[...])


def _dual_matmul(x, wl, wr, b):
    bm = N_NODES // 10
    return pl.pallas_call(
        _mm2_body,
        grid=(10,),
        in_specs=[
            pl.BlockSpec((bm, D), lambda i: (i, 0)),
            pl.BlockSpec((D, D), lambda i: (0, 0)),
            pl.BlockSpec((D, D), lambda i: (0, 0)),
            pl.BlockSpec((1, D), lambda i: (0, 0)),
        ],
        out_specs=[pl.BlockSpec((bm, D), lambda i: (i, 0)),
                   pl.BlockSpec((bm, D), lambda i: (i, 0))],
        out_shape=[jax.ShapeDtypeStruct((NP, D), jnp.float32),
                   jax.ShapeDtypeStruct((NP, D), jnp.float32)],
    )(x, wl, wr, b.reshape(1, D))


def _fused_body(p_ref, z_ref, wl_ref, wr_ref, b_ref, y_ref, z2_ref):
    h = jnp.maximum(p_ref[0] + p_ref[1] + z_ref[...], 0.0)
    y_ref[...] = jnp.dot(h, wl_ref[...], preferred_element_type=jnp.float32)
    z2_ref[...] = (jnp.dot(h, wr_ref[...], preferred_element_type=jnp.float32)
                   + b_ref[...])


def _relu_sum_matmul(p, z1, wl, wr, b):
    return pl.pallas_call(
        _fused_body,
        grid=(NP // BM,),
        in_specs=[
            pl.BlockSpec((2, BM, D), lambda i: (0, i, 0)),
            pl.BlockSpec((BM, D), lambda i: (i, 0)),
            pl.BlockSpec((D, D), lambda i: (0, 0)),
            pl.BlockSpec((D, D), lambda i: (0, 0)),
            pl.BlockSpec((1, D), lambda i: (0, 0)),
        ],
        out_specs=[pl.BlockSpec((BM, D), lambda i: (i, 0)),
                   pl.BlockSpec((BM, D), lambda i: (i, 0))],
        out_shape=[jax.ShapeDtypeStruct((NP, D), jnp.float32),
                   jax.ShapeDtypeStruct((NP, D), jnp.float32)],
    )(p, z1, wl, wr, b.reshape(1, D))


def _sum_body(q_ref, z_ref, o_ref):
    o_ref[...] = q_ref[0] + q_ref[1] + z_ref[...]


def _partial_sum(q, z2):
    return pl.pallas_call(
        _sum_body,
        grid=(NP // BM,),
        in_specs=[
            pl.BlockSpec((2, BM, D), lambda i: (0, i, 0)),
            pl.BlockSpec((BM, D), lambda i: (i, 0)),
        ],
        out_specs=pl.BlockSpec((BM, D), lambda i: (i, 0)),
        out_shape=jax.ShapeDtypeStruct((NP, D), jnp.float32),
    )(q, z2)


_MESH = plsc.VectorSubcoreMesh(core_axis_name="c", subcore_axis_name="s")


@functools.partial(
    pl.kernel,
    out_type=jax.ShapeDtypeStruct((NC, NP, D), jnp.float32),
    mesh=_MESH,
    compiler_params=pltpu.CompilerParams(needs_layout_passes=False),
    scratch_types=[
        pltpu.VMEM_SHARED((NP, D), jnp.float32),
        pltpu.VMEM((KBS // 2, BS), jnp.int32),
        pltpu.VMEM((KBS // 2, BS), jnp.int32),
        pltpu.VMEM((BS, D), jnp.float32),
        pltpu.VMEM((BS, D), jnp.float32),
        pltpu.SemaphoreType.DMA,
        pltpu.SemaphoreType.DMA,
    ],
)
def _scatter_kernel(y_hbm, src_hbm, dst_hbm, out_hbm, acc, sidx, didx,
                    rows_a, rows_b, sem_a, sem_b):
    cid = lax.axis_index("c")
    sid = lax.axis_index("s")
    wid = cid * NS + sid

    def _zero(k, _):
        rows_a[k // 8, pl.ds((k % 8) * 16, 16)] = jnp.zeros((16,), jnp.float32)
        return 0
    lax.fori_loop(0, BS * 8, _zero, 0)
    for i in range(ROWS_PER_TILE // BS):
        pltpu.sync_copy(rows_a,
                        acc.at[pl.ds(sid * ROWS_PER_TILE + i * BS, BS)])

    plsc.subcore_barrier()

    def _start(j, buf, sem):
        pltpu.async_copy(y_hbm.at[sidx.at[j]], buf, sem)

    def _wait(j, buf, sem):
        pltpu.make_async_copy(y_hbm.at[sidx.at[j]], buf, sem).wait()

    hkb = KBS // 2
    for ph in range(2):
        pltpu.sync_copy(src_hbm.at[wid, pl.ds(ph * hkb, hkb)], sidx)
        pltpu.sync_copy(dst_hbm.at[wid, pl.ds(ph * hkb, hkb)], didx)
        _start(0, rows_a, sem_a)

        def _body(k, _):
            j0 = 2 * k
            j1 = j0 + 1
            _start(j1, rows_b, sem_b)
            _wait(j0, rows_a, sem_a)
            pltpu.sync_copy(rows_a, acc.at[didx.at[j0]], add=True)

            @pl.when(k < hkb // 2 - 1)
            def _():
                _start(j1 + 1, rows_a, sem_a)
            _wait(j1, rows_b, sem_b)
            pltpu.sync_copy(rows_b, acc.at[didx.at[j1]], add=True)
            return 0
        lax.fori_loop(0, hkb // 2, _body, 0)

    plsc.subcore_barrier()
    pltpu.sync_copy(acc.at[pl.ds(sid * ROWS_PER_TILE, ROWS_PER_TILE)],
                    out_hbm.at[cid, pl.ds(sid * ROWS_PER_TILE, ROWS_PER_TILE)])


@functools.partial(
    pl.kernel,
    out_type=jax.ShapeDtypeStruct((NE_PAD,), jnp.float32),
    mesh=_MESH,
    compiler_params=pltpu.CompilerParams(needs_layout_passes=False),
    scratch_types=[
        pltpu.VMEM((KB, 128), jnp.int32),
        pltpu.VMEM((KB, 128), jnp.int32),
        pltpu.VMEM((128, D), jnp.float32),
        pltpu.VMEM((128, D), jnp.float32),
        pltpu.VMEM((128, D), jnp.float32),
        pltpu.VMEM((128, D), jnp.float32),
        pltpu.VMEM((256,), jnp.float32),
        pltpu.VMEM((256,), jnp.float32),
        pltpu.VMEM((128,), jnp.float32),
        pltpu.VMEM((128,), jnp.float32),
        pltpu.SemaphoreType.DMA,
        pltpu.SemaphoreType.DMA,
        pltpu.SemaphoreType.DMA,
        pltpu.SemaphoreType.DMA,
    ],
)
def _edgedot_kernel(h_hbm, src_hbm, dst_hbm, out_hbm, sidx, didx, srows_a,
                    drows_a, srows_b, drows_b, tbuf_a, tbuf_b, obuf_a, obuf_b,
                    sem_a, sem_b, sem_oa, sem_ob):
    cid = lax.axis_index("c")
    sid = lax.axis_index("s")
    wid = cid * NS + sid
    pltpu.sync_copy(src_hbm.at[wid], sidx)
    pltpu.sync_copy(dst_hbm.at[wid], didx)
    col0 = lax.iota(jnp.int32, 16) * 16

    def _start(j, sbuf, dbuf, sem):
        pltpu.async_copy(h_hbm.at[sidx.at[j]], sbuf, sem)
        pltpu.async_copy(h_hbm.at[didx.at[j]], dbuf, sem)

    def _wait(j, sbuf, dbuf, sem):
        pltpu.make_async_copy(h_hbm.at[sidx.at[j]], sbuf, sem).wait()
        pltpu.make_async_copy(h_hbm.at[didx.at[j]], dbuf, sem).wait()

    def _owait(j, ob, sem):
        pltpu.make_async_copy(
            ob, out_hbm.at[pl.ds(wid * EW + j * 128, 128)], sem).wait()

    def _products(e0, srows, drows, tref):
        for e16 in range(16):
            e = e0 + e16
            p = [srows[e, pl.ds(c * 16, 16)] * drows[e, pl.ds(c * 16, 16)]
                 for c in range(8)]
            q = [p[0] + p[1], p[2] + p[3], p[4] + p[5], p[6] + p[7]]
            tref[pl.ds(e16 * 16, 16)] = (q[0] + q[1]) + (q[2] + q[3])

    def _reduce(g, tref, ob):
        v = [plsc.load_gather(tref, [col0 + l]) for l in range(16)]
        for step in (8, 4, 2, 1):
            v = [v[i] + v[i + step] for i in range(step)]
        ob[pl.ds(g * 16, 16)] = 1.0 / (1.0 + jnp.exp(-v[0]))

    def _compute(j, srows, drows, ob, sem_o):
        def _gpair(gp, _):
            g0 = 2 * gp
            _products(g0 * 16, srows, drows, tbuf_a)
            _products(g0 * 16 + 16, srows, drows, tbuf_b)
            _reduce(g0, tbuf_a, ob)
            _reduce(g0 + 1, tbuf_b, ob)
            return 0
        lax.fori_loop(0, 4, _gpair, 0)
        pltpu.async_copy(ob, out_hbm.at[pl.ds(wid * EW + j * 128, 128)],
                         sem_o)

    _start(0, srows_a, drows_a, sem_a)

    def _body(k, _):
        j0 = 2 * k
        j1 = j0 + 1
        _start(j1, srows_b, drows_b, sem_b)
        _wait(j0, srows_a, drows_a, sem_a)

        @pl.when(k > 0)
        def _():
            _owait(j0 - 2, obuf_a, sem_oa)
        _compute(j0, srows_a, drows_a, obuf_a, sem_oa)

        @pl.when(k < KB // 2 - 1)
        def _():
            _start(j1 + 1, srows_a, drows_a, sem_a)
        _wait(j1, srows_b, drows_b, sem_b)

        @pl.when(k > 0)
        def _():
            _owait(j1 - 2, obuf_b, sem_ob)
        _compute(j1, srows_b, drows_b, obuf_b, sem_ob)
        return 0
    lax.fori_loop(0, KB // 2, _body, 0)
    _owait(KB - 2, obuf_a, sem_oa)
    _owait(KB - 1, obuf_b, sem_ob)


def _prep_edges(idx2, pad, nb):
    s = jnp.concatenate([idx2[0], pad]).reshape(NW, nb, EW // nb)
    d = jnp.concatenate([idx2[1], pad]).reshape(NW, nb, EW // nb)
    return s, d


def kernel(Features, A, E, Wl1, Wr1, b1, Wl2, Wr2, b2):
    pad = N_NODES + (jnp.arange(NE_PAD - NE, dtype=jnp.int32)
                     % (NP - N_NODES))
    a_src, a_dst = _prep_edges(A, pad, KBS)
    e_src, e_dst = _prep_edges(E, pad, KB)

    y1, z1 = _dual_matmul(Features, Wl1, Wr1, b1)
    p = _scatter_kernel(y1, a_src, a_dst)
    y2, z2 = _relu_sum_matmul(p, z1, Wl2, Wr2, b2)
    q = _scatter_kernel(y2, a_src, a_dst)
    h2 = _partial_sum(q, z2)
    out = _edgedot_kernel(h2, e_src, e_dst)
    return out[:NE]
